# Initial kernel scaffold; baseline (speedup 1.0000x reference)
import functools

import jax
import jax.numpy as jnp
from jax import lax
from jax.experimental import pallas as pl
from jax.experimental.pallas import tpu as pltpu
from jax.experimental.pallas import tpu_sc as plsc

T = 2048
H = 1024
E = 16
I = 2816
IT = 256
NI = I // IT
BLK = 128
NBLK = 48
NROWS = NBLK * BLK
NC = 2
NS = 16
NW = NC * NS
TPW = T // NW
CH = 32


def _routing_body(logits_ref, p0_ref, p1_ref, ws0_ref, ws1_ref, be_ref):
    lg = logits_ref[...]
    col = lax.broadcasted_iota(jnp.int32, (T, E), 1)
    big = jnp.int32(10**9)
    m1 = jnp.max(lg, axis=1, keepdims=True)
    a1 = jnp.min(jnp.where(lg == m1, col, big), axis=1, keepdims=True)
    lg2 = jnp.where(col == a1, jnp.float32(-1e30), lg)
    m2 = jnp.max(lg2, axis=1, keepdims=True)
    a2 = jnp.min(jnp.where(lg2 == m2, col, big), axis=1, keepdims=True)
    w0 = jax.nn.sigmoid(m1 - m2)
    w1 = 1.0 - w0

    oh = (col == a1).astype(jnp.float32) + (col == a2).astype(jnp.float32)
    r128 = lax.broadcasted_iota(jnp.int32, (128, 128), 0)
    c128 = lax.broadcasted_iota(jnp.int32, (128, 128), 1)
    tril = (r128 > c128).astype(jnp.float32)
    chunks = []
    carry = jnp.zeros((1, E), jnp.float32)
    for k in range(T // 128):
        ch = oh[k * 128:(k + 1) * 128, :]
        chunks.append(jnp.dot(tril, ch, preferred_element_type=jnp.float32) + carry)
        carry = carry + jnp.sum(ch, axis=0, keepdims=True)
    cum = jnp.concatenate(chunks, axis=0)
    counts = carry
    nblk_e = jnp.floor((counts + 127.0) / 128.0)
    r16 = lax.broadcasted_iota(jnp.int32, (E, E), 0)
    c16 = lax.broadcasted_iota(jnp.int32, (E, E), 1)
    upper = (r16 < c16).astype(jnp.float32)
    sb = jnp.dot(nblk_e, upper, preferred_element_type=jnp.float32)
    start = sb * float(BLK)
    pos = jnp.broadcast_to(start, (T, E)) + cum
    p0 = jnp.sum(jnp.where(col == a1, pos, 0.0), axis=1, keepdims=True)
    p1 = jnp.sum(jnp.where(col == a2, pos, 0.0), axis=1, keepdims=True)
    p0_ref[...] = p0.astype(jnp.int32)
    p1_ref[...] = p1.astype(jnp.int32)
    ws0_ref[...] = jnp.broadcast_to(w0, (T, E))
    ws1_ref[...] = jnp.broadcast_to(w1, (T, E))
    bio = lax.broadcasted_iota(jnp.float32, (64, E), 0)
    cnt = jnp.sum((bio >= jnp.broadcast_to(sb, (64, E))).astype(jnp.int32),
                  axis=1, keepdims=True)
    be_ref[...] = jnp.clip(cnt - 1, 0, E - 1)


def _routing(router_logits):
    return pl.pallas_call(
        _routing_body,
        out_shape=[
            jax.ShapeDtypeStruct((T, 1), jnp.int32),
            jax.ShapeDtypeStruct((T, 1), jnp.int32),
            jax.ShapeDtypeStruct((T, E), jnp.float32),
            jax.ShapeDtypeStruct((T, E), jnp.float32),
            jax.ShapeDtypeStruct((64, 1), jnp.int32),
        ],
    )(router_logits)


def _dispatch(x, p0, p1):
    mesh = plsc.VectorSubcoreMesh(core_axis_name="c", subcore_axis_name="s")

    @functools.partial(
        pl.kernel,
        mesh=mesh,
        out_type=jax.ShapeDtypeStruct((NROWS, H), jnp.float32),
        scratch_types=[
            pltpu.VMEM((TPW,), jnp.int32),
            pltpu.VMEM((TPW,), jnp.int32),
            pltpu.VMEM((TPW, H), jnp.float32),
        ],
    )
    def k(x_hbm, p0_hbm, p1_hbm, xs_hbm, i0_v, i1_v, rows_v):
        wid = lax.axis_index("s") * NC + lax.axis_index("c")
        base = pl.multiple_of(wid * TPW, TPW)
        pltpu.sync_copy(p0_hbm.at[pl.ds(base, TPW)], i0_v)
        pltpu.sync_copy(p1_hbm.at[pl.ds(base, TPW)], i1_v)
        pltpu.sync_copy(x_hbm.at[pl.ds(base, TPW)], rows_v)
        pltpu.sync_copy(rows_v, xs_hbm.at[i0_v])
        pltpu.sync_copy(rows_v, xs_hbm.at[i1_v])

    return k(x, p0, p1)


def _mm_body(be_ref, xs_ref, wg_ref, wu_ref, w2_ref, ys_ref, acc_ref):
    i = pl.program_id(0)
    b = pl.program_id(1)
    x = xs_ref[...]
    g = jnp.dot(x, wg_ref[0], preferred_element_type=jnp.float32)
    u = jnp.dot(x, wu_ref[0], preferred_element_type=jnp.float32)
    h = g * jax.nn.sigmoid(g) * u
    contrib = jnp.dot(h, w2_ref[0], preferred_element_type=jnp.float32)
    sl = pl.ds(b * BLK, BLK)

    @pl.when(i == 0)
    def _():
        acc_ref[sl, :] = contrib

    @pl.when(i > 0)
    def _():
        acc_ref[sl, :] += contrib

    @pl.when(i == NI - 1)
    def _():
        ys_ref[...] = acc_ref[sl, :]


def _grouped_mm(be, xs, w13, w2):
    grid_spec = pltpu.PrefetchScalarGridSpec(
        num_scalar_prefetch=1,
        grid=(NI, NBLK),
        in_specs=[
            pl.BlockSpec((BLK, H), lambda i, b, be: (b, 0)),
            pl.BlockSpec((1, H, IT), lambda i, b, be: (be[b], 0, i)),
            pl.BlockSpec((1, H, IT), lambda i, b, be: (be[b], 0, NI + i)),
            pl.BlockSpec((1, IT, H), lambda i, b, be: (be[b], i, 0)),
        ],
        out_specs=pl.BlockSpec(
            (BLK, H), lambda i, b, be: (jnp.where(i == NI - 1, b, 0), 0)),
        scratch_shapes=[pltpu.VMEM((NROWS, H), jnp.float32)],
    )
    return pl.pallas_call(
        _mm_body,
        grid_spec=grid_spec,
        out_shape=jax.ShapeDtypeStruct((NROWS, H), jnp.float32),
        compiler_params=pltpu.CompilerParams(
            dimension_semantics=("arbitrary", "arbitrary")),
    )(be, xs, w13, w13, w2)


def _combine(ys, p0, p1, ws0, ws1):
    mesh = plsc.VectorSubcoreMesh(core_axis_name="c", subcore_axis_name="s")

    @functools.partial(
        pl.kernel,
        mesh=mesh,
        out_type=jax.ShapeDtypeStruct((T, H), jnp.float32),
        scratch_types=[
            pltpu.VMEM((CH,), jnp.int32),
            pltpu.VMEM((CH,), jnp.int32),
            pltpu.VMEM((CH, H), jnp.float32),
            pltpu.VMEM((CH, H), jnp.float32),
            pltpu.VMEM((CH, E), jnp.float32),
            pltpu.VMEM((CH, E), jnp.float32),
        ],
    )
    def k(ys_hbm, p0_hbm, p1_hbm, ws0_hbm, ws1_hbm, out_hbm,
          i0_v, i1_v, r0_v, r1_v, w0_v, w1_v):
        wid = lax.axis_index("s") * NC + lax.axis_index("c")
        base = pl.multiple_of(wid * TPW, TPW)

        @pl.loop(0, TPW // CH)
        def _(cix):
            cbase = pl.multiple_of(base + cix * CH, CH)
            pltpu.sync_copy(p0_hbm.at[pl.ds(cbase, CH)], i0_v)
            pltpu.sync_copy(p1_hbm.at[pl.ds(cbase, CH)], i1_v)
            pltpu.sync_copy(ws0_hbm.at[pl.ds(cbase, CH)], w0_v)
            pltpu.sync_copy(ws1_hbm.at[pl.ds(cbase, CH)], w1_v)
            pltpu.sync_copy(ys_hbm.at[i0_v], r0_v)
            pltpu.sync_copy(ys_hbm.at[i1_v], r1_v)

            @pl.loop(0, CH)
            def _(r):
                w0c = w0_v[r]
                w1c = w1_v[r]

                @pl.loop(0, H // E)
                def _(c):
                    slc = pl.ds(c * E, E)
                    r0_v[r, slc] = r0_v[r, slc] * w0c + r1_v[r, slc] * w1c

            pltpu.sync_copy(r0_v, out_hbm.at[pl.ds(cbase, CH)])

    return k(ys, p0, p1, ws0, ws1)


def kernel(hidden_states, router_logits, w13, w2):
    p0c, p1c, ws0, ws1, bec = _routing(router_logits)
    p0 = p0c.reshape(T)
    p1 = p1c.reshape(T)
    be = bec.reshape(64)[:NBLK]
    xs = _dispatch(hidden_states, p0, p1)
    ys = _grouped_mm(be, xs, w13, w2)
    return _combine(ys, p0, p1, ws0, ws1)


# speedup vs baseline: 1.5101x; 1.5101x over previous
import functools

import jax
import jax.numpy as jnp
from jax import lax
from jax.experimental import pallas as pl
from jax.experimental.pallas import tpu as pltpu
from jax.experimental.pallas import tpu_sc as plsc

T = 2048
H = 1024
E = 16
I = 2816
IT = 256
NI = I // IT
BLK = 128
NBLK = 48
NROWS = NBLK * BLK
NC = 2
NS = 16
NW = NC * NS
TPW = T // NW
CH = 32


def _routing_body(logits_ref, p0_ref, p1_ref, ws0_ref, ws1_ref, be_ref):
    lg = logits_ref[...]
    col = lax.broadcasted_iota(jnp.int32, (T, E), 1)
    big = jnp.int32(10**9)
    m1 = jnp.max(lg, axis=1, keepdims=True)
    a1 = jnp.min(jnp.where(lg == m1, col, big), axis=1, keepdims=True)
    lg2 = jnp.where(col == a1, jnp.float32(-1e30), lg)
    m2 = jnp.max(lg2, axis=1, keepdims=True)
    a2 = jnp.min(jnp.where(lg2 == m2, col, big), axis=1, keepdims=True)
    w0 = jax.nn.sigmoid(m1 - m2)
    w1 = 1.0 - w0

    oh = (col == a1).astype(jnp.float32) + (col == a2).astype(jnp.float32)
    r128 = lax.broadcasted_iota(jnp.int32, (128, 128), 0)
    c128 = lax.broadcasted_iota(jnp.int32, (128, 128), 1)
    tril = (r128 > c128).astype(jnp.float32)
    chunks = []
    carry = jnp.zeros((1, E), jnp.float32)
    for k in range(T // 128):
        ch = oh[k * 128:(k + 1) * 128, :]
        chunks.append(jnp.dot(tril, ch, preferred_element_type=jnp.float32) + carry)
        carry = carry + jnp.sum(ch, axis=0, keepdims=True)
    cum = jnp.concatenate(chunks, axis=0)
    counts = carry
    nblk_e = jnp.floor((counts + 127.0) / 128.0)
    r16 = lax.broadcasted_iota(jnp.int32, (E, E), 0)
    c16 = lax.broadcasted_iota(jnp.int32, (E, E), 1)
    upper = (r16 < c16).astype(jnp.float32)
    sb = jnp.dot(nblk_e, upper, preferred_element_type=jnp.float32)
    start = sb * float(BLK)
    pos = jnp.broadcast_to(start, (T, E)) + cum
    p0 = jnp.sum(jnp.where(col == a1, pos, 0.0), axis=1, keepdims=True)
    p1 = jnp.sum(jnp.where(col == a2, pos, 0.0), axis=1, keepdims=True)
    p0_ref[...] = p0.astype(jnp.int32)
    p1_ref[...] = p1.astype(jnp.int32)
    ws0_ref[...] = jnp.broadcast_to(w0, (T, E))
    ws1_ref[...] = jnp.broadcast_to(w1, (T, E))
    bio = lax.broadcasted_iota(jnp.int32, (64, E), 0).astype(jnp.float32)
    cnt = jnp.sum((bio >= jnp.broadcast_to(sb, (64, E))).astype(jnp.int32),
                  axis=1, keepdims=True)
    be_ref[...] = jnp.clip(cnt - 1, 0, E - 1)


def _routing(router_logits):
    return pl.pallas_call(
        _routing_body,
        out_shape=[
            jax.ShapeDtypeStruct((T, 1), jnp.int32),
            jax.ShapeDtypeStruct((T, 1), jnp.int32),
            jax.ShapeDtypeStruct((T, E), jnp.float32),
            jax.ShapeDtypeStruct((T, E), jnp.float32),
            jax.ShapeDtypeStruct((64, 1), jnp.int32),
        ],
    )(router_logits)


def _dispatch(x, p0, p1):
    mesh = plsc.VectorSubcoreMesh(core_axis_name="c", subcore_axis_name="s")

    @functools.partial(
        pl.kernel,
        mesh=mesh,
        out_type=jax.ShapeDtypeStruct((NROWS, H), jnp.float32),
        scratch_types=[
            pltpu.VMEM((TPW,), jnp.int32),
            pltpu.VMEM((TPW,), jnp.int32),
            pltpu.VMEM((TPW, H), jnp.float32),
        ],
    )
    def k(x_hbm, p0_hbm, p1_hbm, xs_hbm, i0_v, i1_v, rows_v):
        wid = lax.axis_index("s") * NC + lax.axis_index("c")
        base = pl.multiple_of(wid * TPW, TPW)
        pltpu.sync_copy(p0_hbm.at[pl.ds(base, TPW)], i0_v)
        pltpu.sync_copy(p1_hbm.at[pl.ds(base, TPW)], i1_v)
        pltpu.sync_copy(x_hbm.at[pl.ds(base, TPW)], rows_v)
        pltpu.sync_copy(rows_v, xs_hbm.at[i0_v])
        pltpu.sync_copy(rows_v, xs_hbm.at[i1_v])

    return k(x, p0, p1)


def _mm_body(be_ref, xs_ref, wg_ref, wu_ref, w2_ref, ys_ref, acc_ref):
    i = pl.program_id(0)
    b = pl.program_id(1)
    x = xs_ref[...]
    g = jnp.dot(x, wg_ref[0], preferred_element_type=jnp.float32)
    u = jnp.dot(x, wu_ref[0], preferred_element_type=jnp.float32)
    h = g * jax.nn.sigmoid(g) * u
    contrib = jnp.dot(h, w2_ref[0], preferred_element_type=jnp.float32)
    sl = pl.ds(b * BLK, BLK)

    @pl.when(i == 0)
    def _():
        acc_ref[sl, :] = contrib

    @pl.when(i > 0)
    def _():
        acc_ref[sl, :] += contrib

    @pl.when(i == NI - 1)
    def _():
        ys_ref[...] = acc_ref[sl, :]


def _grouped_mm(be, xs, w13, w2):
    grid_spec = pltpu.PrefetchScalarGridSpec(
        num_scalar_prefetch=1,
        grid=(NI, NBLK),
        in_specs=[
            pl.BlockSpec((BLK, H), lambda i, b, be: (b, 0)),
            pl.BlockSpec((1, H, IT), lambda i, b, be: (be[b], 0, i)),
            pl.BlockSpec((1, H, IT), lambda i, b, be: (be[b], 0, NI + i)),
            pl.BlockSpec((1, IT, H), lambda i, b, be: (be[b], i, 0)),
        ],
        out_specs=pl.BlockSpec(
            (BLK, H), lambda i, b, be: (jnp.where(i == NI - 1, b, 0), 0)),
        scratch_shapes=[pltpu.VMEM((NROWS, H), jnp.float32)],
    )
    return pl.pallas_call(
        _mm_body,
        grid_spec=grid_spec,
        out_shape=jax.ShapeDtypeStruct((NROWS, H), jnp.float32),
        compiler_params=pltpu.CompilerParams(
            dimension_semantics=("arbitrary", "arbitrary")),
    )(be, xs, w13, w13, w2)


def _combine(ys, p0, p1, ws0, ws1):
    mesh = plsc.VectorSubcoreMesh(core_axis_name="c", subcore_axis_name="s")

    @functools.partial(
        pl.kernel,
        mesh=mesh,
        out_type=jax.ShapeDtypeStruct((T, H), jnp.float32),
        scratch_types=[
            pltpu.VMEM((CH,), jnp.int32),
            pltpu.VMEM((CH,), jnp.int32),
            pltpu.VMEM((CH, H), jnp.float32),
            pltpu.VMEM((CH, H), jnp.float32),
            pltpu.VMEM((CH, E), jnp.float32),
            pltpu.VMEM((CH, E), jnp.float32),
        ],
    )
    def k(ys_hbm, p0_hbm, p1_hbm, ws0_hbm, ws1_hbm, out_hbm,
          i0_v, i1_v, r0_v, r1_v, w0_v, w1_v):
        wid = lax.axis_index("s") * NC + lax.axis_index("c")
        base = pl.multiple_of(wid * TPW, TPW)

        @pl.loop(0, TPW // CH)
        def _(cix):
            cbase = pl.multiple_of(base + cix * CH, CH)
            pltpu.sync_copy(p0_hbm.at[pl.ds(cbase, CH)], i0_v)
            pltpu.sync_copy(p1_hbm.at[pl.ds(cbase, CH)], i1_v)
            pltpu.sync_copy(ws0_hbm.at[pl.ds(cbase, CH)], w0_v)
            pltpu.sync_copy(ws1_hbm.at[pl.ds(cbase, CH)], w1_v)
            pltpu.sync_copy(ys_hbm.at[i0_v], r0_v)
            pltpu.sync_copy(ys_hbm.at[i1_v], r1_v)

            @pl.loop(0, CH)
            def _(r):
                w0c = w0_v[r]
                w1c = w1_v[r]

                @pl.loop(0, H // E)
                def _(c):
                    slc = pl.ds(c * E, E)
                    r0_v[r, slc] = r0_v[r, slc] * w0c + r1_v[r, slc] * w1c

            pltpu.sync_copy(r0_v, out_hbm.at[pl.ds(cbase, CH)])

    return k(ys, p0, p1, ws0, ws1)


def kernel(hidden_states, router_logits, w13, w2):
    p0c, p1c, ws0, ws1, bec = _routing(router_logits)
    p0 = p0c.reshape(T)
    p1 = p1c.reshape(T)
    be = bec.reshape(64)[:NBLK]
    xs = _dispatch(hidden_states, p0, p1)
    ys = _grouped_mm(be, xs, w13, w2)
    return _combine(ys, p0, p1, ws0, ws1)


# speedup vs baseline: 1.5550x; 1.0297x over previous
import functools

import jax
import jax.numpy as jnp
from jax import lax
from jax.experimental import pallas as pl
from jax.experimental.pallas import tpu as pltpu
from jax.experimental.pallas import tpu_sc as plsc

T = 2048
H = 1024
E = 16
I = 2816
IT = 256
NI = I // IT
BLK = 128
NBLK = 48
NROWS = NBLK * BLK
NC = 2
NS = 16
NW = NC * NS
TPW = T // NW
CH = 32


def _routing_body(logits_ref, p0_ref, p1_ref, ws0_ref, ws1_ref, be_ref):
    lg = logits_ref[...]
    col = lax.broadcasted_iota(jnp.int32, (T, E), 1)
    big = jnp.int32(10**9)
    m1 = jnp.max(lg, axis=1, keepdims=True)
    a1 = jnp.min(jnp.where(lg == m1, col, big), axis=1, keepdims=True)
    lg2 = jnp.where(col == a1, jnp.float32(-1e30), lg)
    m2 = jnp.max(lg2, axis=1, keepdims=True)
    a2 = jnp.min(jnp.where(lg2 == m2, col, big), axis=1, keepdims=True)
    w0 = jax.nn.sigmoid(m1 - m2)
    w1 = 1.0 - w0

    oh = (col == a1).astype(jnp.float32) + (col == a2).astype(jnp.float32)
    r128 = lax.broadcasted_iota(jnp.int32, (128, 128), 0)
    c128 = lax.broadcasted_iota(jnp.int32, (128, 128), 1)
    tril = (r128 > c128).astype(jnp.float32)
    chunks = []
    carry = jnp.zeros((1, E), jnp.float32)
    for k in range(T // 128):
        ch = oh[k * 128:(k + 1) * 128, :]
        chunks.append(jnp.dot(tril, ch, preferred_element_type=jnp.float32) + carry)
        carry = carry + jnp.sum(ch, axis=0, keepdims=True)
    cum = jnp.concatenate(chunks, axis=0)
    counts = carry
    nblk_e = jnp.floor((counts + 127.0) / 128.0)
    r16 = lax.broadcasted_iota(jnp.int32, (E, E), 0)
    c16 = lax.broadcasted_iota(jnp.int32, (E, E), 1)
    upper = (r16 < c16).astype(jnp.float32)
    sb = jnp.dot(nblk_e, upper, preferred_element_type=jnp.float32)
    start = sb * float(BLK)
    pos = jnp.broadcast_to(start, (T, E)) + cum
    p0 = jnp.sum(jnp.where(col == a1, pos, 0.0), axis=1, keepdims=True)
    p1 = jnp.sum(jnp.where(col == a2, pos, 0.0), axis=1, keepdims=True)
    p0_ref[...] = p0.astype(jnp.int32)
    p1_ref[...] = p1.astype(jnp.int32)
    ws0_ref[...] = jnp.broadcast_to(w0, (T, E))
    ws1_ref[...] = jnp.broadcast_to(w1, (T, E))
    bio = lax.broadcasted_iota(jnp.int32, (64, E), 0).astype(jnp.float32)
    cnt = jnp.sum((bio >= jnp.broadcast_to(sb, (64, E))).astype(jnp.int32),
                  axis=1, keepdims=True)
    be_ref[...] = jnp.clip(cnt - 1, 0, E - 1)


def _routing(router_logits):
    return pl.pallas_call(
        _routing_body,
        out_shape=[
            jax.ShapeDtypeStruct((T, 1), jnp.int32),
            jax.ShapeDtypeStruct((T, 1), jnp.int32),
            jax.ShapeDtypeStruct((T, E), jnp.float32),
            jax.ShapeDtypeStruct((T, E), jnp.float32),
            jax.ShapeDtypeStruct((64, 1), jnp.int32),
        ],
    )(router_logits)


def _dispatch(x, p0, p1):
    mesh = plsc.VectorSubcoreMesh(core_axis_name="c", subcore_axis_name="s")

    @functools.partial(
        pl.kernel,
        mesh=mesh,
        out_type=jax.ShapeDtypeStruct((NROWS, H), jnp.float32),
        scratch_types=[
            pltpu.VMEM((TPW,), jnp.int32),
            pltpu.VMEM((TPW,), jnp.int32),
            pltpu.VMEM((TPW, H), jnp.float32),
        ],
    )
    def k(x_hbm, p0_hbm, p1_hbm, xs_hbm, i0_v, i1_v, rows_v):
        wid = lax.axis_index("s") * NC + lax.axis_index("c")
        base = pl.multiple_of(wid * TPW, TPW)
        pltpu.sync_copy(p0_hbm.at[pl.ds(base, TPW)], i0_v)
        pltpu.sync_copy(p1_hbm.at[pl.ds(base, TPW)], i1_v)
        pltpu.sync_copy(x_hbm.at[pl.ds(base, TPW)], rows_v)
        pltpu.sync_copy(rows_v, xs_hbm.at[i0_v])
        pltpu.sync_copy(rows_v, xs_hbm.at[i1_v])

    return k(x, p0, p1)


def _mm_body(be_ref, xs_ref, wg_ref, wu_ref, w2_ref, ys_ref, acc_ref, xc_ref):
    i = pl.program_id(0)
    b = pl.program_id(1)
    sl = pl.ds(b * BLK, BLK)

    @pl.when(i == 0)
    def _():
        xc_ref[sl, :] = xs_ref[...].astype(jnp.bfloat16)

    x = xc_ref[sl, :]
    g = jnp.dot(x, wg_ref[0].astype(jnp.bfloat16),
                preferred_element_type=jnp.float32)
    u = jnp.dot(x, wu_ref[0].astype(jnp.bfloat16),
                preferred_element_type=jnp.float32)
    h = (g * jax.nn.sigmoid(g) * u).astype(jnp.bfloat16)
    contrib = jnp.dot(h, w2_ref[0].astype(jnp.bfloat16),
                      preferred_element_type=jnp.float32)

    @pl.when(i == 0)
    def _():
        acc_ref[sl, :] = contrib

    @pl.when(i > 0)
    def _():
        acc_ref[sl, :] += contrib

    @pl.when(i == NI - 1)
    def _():
        ys_ref[...] = acc_ref[sl, :]


def _grouped_mm(be, xs, w13, w2):
    grid_spec = pltpu.PrefetchScalarGridSpec(
        num_scalar_prefetch=1,
        grid=(NI, NBLK),
        in_specs=[
            pl.BlockSpec((BLK, H), lambda i, b, be: (jnp.where(i == 0, b, 0), 0)),
            pl.BlockSpec((1, H, IT), lambda i, b, be: (be[b], 0, i)),
            pl.BlockSpec((1, H, IT), lambda i, b, be: (be[b], 0, NI + i)),
            pl.BlockSpec((1, IT, H), lambda i, b, be: (be[b], i, 0)),
        ],
        out_specs=pl.BlockSpec(
            (BLK, H), lambda i, b, be: (jnp.where(i == NI - 1, b, 0), 0)),
        scratch_shapes=[pltpu.VMEM((NROWS, H), jnp.float32),
                        pltpu.VMEM((NROWS, H), jnp.bfloat16)],
    )
    return pl.pallas_call(
        _mm_body,
        grid_spec=grid_spec,
        out_shape=jax.ShapeDtypeStruct((NROWS, H), jnp.float32),
        compiler_params=pltpu.CompilerParams(
            dimension_semantics=("arbitrary", "arbitrary")),
    )(be, xs, w13, w13, w2)


def _combine(ys, p0, p1, ws0, ws1):
    mesh = plsc.VectorSubcoreMesh(core_axis_name="c", subcore_axis_name="s")

    @functools.partial(
        pl.kernel,
        mesh=mesh,
        out_type=jax.ShapeDtypeStruct((T, H), jnp.float32),
        scratch_types=[
            pltpu.VMEM((CH,), jnp.int32),
            pltpu.VMEM((CH,), jnp.int32),
            pltpu.VMEM((CH, H), jnp.float32),
            pltpu.VMEM((CH, H), jnp.float32),
            pltpu.VMEM((CH, E), jnp.float32),
            pltpu.VMEM((CH, E), jnp.float32),
        ],
    )
    def k(ys_hbm, p0_hbm, p1_hbm, ws0_hbm, ws1_hbm, out_hbm,
          i0_v, i1_v, r0_v, r1_v, w0_v, w1_v):
        wid = lax.axis_index("s") * NC + lax.axis_index("c")
        base = pl.multiple_of(wid * TPW, TPW)

        @pl.loop(0, TPW // CH)
        def _(cix):
            cbase = pl.multiple_of(base + cix * CH, CH)
            pltpu.sync_copy(p0_hbm.at[pl.ds(cbase, CH)], i0_v)
            pltpu.sync_copy(p1_hbm.at[pl.ds(cbase, CH)], i1_v)
            pltpu.sync_copy(ws0_hbm.at[pl.ds(cbase, CH)], w0_v)
            pltpu.sync_copy(ws1_hbm.at[pl.ds(cbase, CH)], w1_v)
            pltpu.sync_copy(ys_hbm.at[i0_v], r0_v)
            pltpu.sync_copy(ys_hbm.at[i1_v], r1_v)

            @pl.loop(0, CH)
            def _(r):
                w0c = w0_v[r]
                w1c = w1_v[r]

                @pl.loop(0, H // E)
                def _(c):
                    slc = pl.ds(c * E, E)
                    r0_v[r, slc] = r0_v[r, slc] * w0c + r1_v[r, slc] * w1c

            pltpu.sync_copy(r0_v, out_hbm.at[pl.ds(cbase, CH)])

    return k(ys, p0, p1, ws0, ws1)


def kernel(hidden_states, router_logits, w13, w2):
    p0c, p1c, ws0, ws1, bec = _routing(router_logits)
    p0 = p0c.reshape(T)
    p1 = p1c.reshape(T)
    be = bec.reshape(64)[:NBLK]
    xs = _dispatch(hidden_states, p0, p1)
    ys = _grouped_mm(be, xs, w13, w2)
    return _combine(ys, p0, p1, ws0, ws1)


# speedup vs baseline: 2.3745x; 1.5270x over previous
import functools

import jax
import jax.numpy as jnp
from jax import lax
from jax.experimental import pallas as pl
from jax.experimental.pallas import tpu as pltpu
from jax.experimental.pallas import tpu_sc as plsc

T = 2048
H = 1024
E = 16
I = 2816
IT = 256
NI = I // IT
BLK = 128
NBLK = 48
NROWS = NBLK * BLK
NC = 2
NS = 16
NW = NC * NS
TPW = T // NW
CH = 32


def _routing_body(logits_ref, p0_ref, p1_ref, ws0_ref, ws1_ref, sb_ref, nb_ref):
    lg = logits_ref[...]
    col = lax.broadcasted_iota(jnp.int32, (T, E), 1)
    big = jnp.int32(10**9)
    m1 = jnp.max(lg, axis=1, keepdims=True)
    a1 = jnp.min(jnp.where(lg == m1, col, big), axis=1, keepdims=True)
    lg2 = jnp.where(col == a1, jnp.float32(-1e30), lg)
    m2 = jnp.max(lg2, axis=1, keepdims=True)
    a2 = jnp.min(jnp.where(lg2 == m2, col, big), axis=1, keepdims=True)
    w0 = jax.nn.sigmoid(m1 - m2)
    w1 = 1.0 - w0

    oh = (col == a1).astype(jnp.float32) + (col == a2).astype(jnp.float32)
    r128 = lax.broadcasted_iota(jnp.int32, (128, 128), 0)
    c128 = lax.broadcasted_iota(jnp.int32, (128, 128), 1)
    tril = (r128 > c128).astype(jnp.float32)
    chunks = []
    carry = jnp.zeros((1, E), jnp.float32)
    for k in range(T // 128):
        ch = oh[k * 128:(k + 1) * 128, :]
        chunks.append(jnp.dot(tril, ch, preferred_element_type=jnp.float32) + carry)
        carry = carry + jnp.sum(ch, axis=0, keepdims=True)
    cum = jnp.concatenate(chunks, axis=0)
    counts = carry
    nblk_e = jnp.floor((counts + 127.0) / 128.0)
    r16 = lax.broadcasted_iota(jnp.int32, (E, E), 0)
    c16 = lax.broadcasted_iota(jnp.int32, (E, E), 1)
    upper = (r16 < c16).astype(jnp.float32)
    sb = jnp.dot(nblk_e, upper, preferred_element_type=jnp.float32)
    start = sb * float(BLK)
    pos = jnp.broadcast_to(start, (T, E)) + cum
    p0 = jnp.sum(jnp.where(col == a1, pos, 0.0), axis=1, keepdims=True)
    p1 = jnp.sum(jnp.where(col == a2, pos, 0.0), axis=1, keepdims=True)
    p0_ref[...] = p0.astype(jnp.int32)
    p1_ref[...] = p1.astype(jnp.int32)
    ws0_ref[...] = jnp.broadcast_to(w0, (T, E))
    ws1_ref[...] = jnp.broadcast_to(w1, (T, E))
    sb_ref[...] = jnp.broadcast_to(sb.astype(jnp.int32), (8, E))
    nb_ref[...] = jnp.broadcast_to(nblk_e.astype(jnp.int32), (8, E))


def _routing(router_logits):
    return pl.pallas_call(
        _routing_body,
        out_shape=[
            jax.ShapeDtypeStruct((T, 1), jnp.int32),
            jax.ShapeDtypeStruct((T, 1), jnp.int32),
            jax.ShapeDtypeStruct((T, E), jnp.float32),
            jax.ShapeDtypeStruct((T, E), jnp.float32),
            jax.ShapeDtypeStruct((8, E), jnp.int32),
            jax.ShapeDtypeStruct((8, E), jnp.int32),
        ],
    )(router_logits)


def _dispatch(x, p0, p1):
    mesh = plsc.VectorSubcoreMesh(core_axis_name="c", subcore_axis_name="s")

    @functools.partial(
        pl.kernel,
        mesh=mesh,
        out_type=jax.ShapeDtypeStruct((NROWS, H), jnp.float32),
        scratch_types=[
            pltpu.VMEM((TPW,), jnp.int32),
            pltpu.VMEM((TPW,), jnp.int32),
            pltpu.VMEM((TPW, H), jnp.float32),
        ],
    )
    def k(x_hbm, p0_hbm, p1_hbm, xs_hbm, i0_v, i1_v, rows_v):
        wid = lax.axis_index("s") * NC + lax.axis_index("c")
        base = pl.multiple_of(wid * TPW, TPW)
        pltpu.sync_copy(p0_hbm.at[pl.ds(base, TPW)], i0_v)
        pltpu.sync_copy(p1_hbm.at[pl.ds(base, TPW)], i1_v)
        pltpu.sync_copy(x_hbm.at[pl.ds(base, TPW)], rows_v)
        pltpu.sync_copy(rows_v, xs_hbm.at[i0_v])
        pltpu.sync_copy(rows_v, xs_hbm.at[i1_v])

    return k(x, p0, p1)


LCH = 8
LCR = NROWS // LCH
MAXB = T // BLK


def _mm_body(sb_ref, nb_ref, xs_ref, wg_ref, wu_ref, w2_ref, ys_ref,
             xc_ref, acc_ref, tmp_ref, lsem, fsem):
    e = pl.program_id(0)
    i = pl.program_id(1)

    @pl.when((e == 0) & (i == 0))
    def _():
        pltpu.make_async_copy(
            xs_ref.at[pl.ds(0, LCR), :], tmp_ref.at[0], lsem).start()
        for c in range(LCH):
            pltpu.make_async_copy(
                xs_ref.at[pl.ds(c * LCR, LCR), :], tmp_ref.at[c % 2],
                lsem).wait()
            if c + 1 < LCH:
                pltpu.make_async_copy(
                    xs_ref.at[pl.ds((c + 1) * LCR, LCR), :],
                    tmp_ref.at[(c + 1) % 2], lsem).start()
            xc_ref[pl.ds(c * LCR, LCR), :] = tmp_ref[c % 2].astype(jnp.bfloat16)

    s0 = sb_ref[e]
    nb = nb_ref[e]
    wgt = wg_ref[0].astype(jnp.bfloat16)
    wut = wu_ref[0].astype(jnp.bfloat16)
    w2t = w2_ref[0].astype(jnp.bfloat16)

    def blk(j, carry):
        x = xc_ref[pl.ds((s0 + j) * BLK, BLK), :]
        g = jnp.dot(x, wgt, preferred_element_type=jnp.float32)
        u = jnp.dot(x, wut, preferred_element_type=jnp.float32)
        h = (g * jax.nn.sigmoid(g) * u).astype(jnp.bfloat16)
        contrib = jnp.dot(h, w2t, preferred_element_type=jnp.float32)
        asl = pl.ds(j * BLK, BLK)

        @pl.when(i == 0)
        def _():
            acc_ref[asl, :] = contrib

        @pl.when(i > 0)
        def _():
            acc_ref[asl, :] += contrib

        return carry

    lax.fori_loop(0, nb, blk, 0)

    @pl.when(i == NI - 1)
    def _():
        def flush(j, carry):
            pltpu.make_async_copy(
                acc_ref.at[pl.ds(j * BLK, BLK), :],
                ys_ref.at[pl.ds((s0 + j) * BLK, BLK), :], fsem).start()
            return carry

        lax.fori_loop(0, nb, flush, 0)

        def drain(j, carry):
            pltpu.make_async_copy(
                acc_ref.at[pl.ds(0, BLK), :],
                ys_ref.at[pl.ds(0, BLK), :], fsem).wait()
            return carry

        lax.fori_loop(0, nb, drain, 0)


def _grouped_mm(sbr, nbe, xs, w13, w2):
    grid_spec = pltpu.PrefetchScalarGridSpec(
        num_scalar_prefetch=2,
        grid=(E, NI),
        in_specs=[
            pl.BlockSpec(memory_space=pl.ANY),
            pl.BlockSpec((1, H, IT), lambda e, i, sb, nb: (e, 0, i)),
            pl.BlockSpec((1, H, IT), lambda e, i, sb, nb: (e, 0, NI + i)),
            pl.BlockSpec((1, IT, H), lambda e, i, sb, nb: (e, i, 0)),
        ],
        out_specs=pl.BlockSpec(memory_space=pl.ANY),
        scratch_shapes=[
            pltpu.VMEM((NROWS, H), jnp.bfloat16),
            pltpu.VMEM((T, H), jnp.float32),
            pltpu.VMEM((2, LCR, H), jnp.float32),
            pltpu.SemaphoreType.DMA,
            pltpu.SemaphoreType.DMA,
        ],
    )
    return pl.pallas_call(
        _mm_body,
        grid_spec=grid_spec,
        out_shape=jax.ShapeDtypeStruct((NROWS, H), jnp.float32),
        compiler_params=pltpu.CompilerParams(
            dimension_semantics=("arbitrary", "arbitrary")),
    )(sbr, nbe, xs, w13, w13, w2)


def _combine(ys, p0, p1, ws0, ws1):
    mesh = plsc.VectorSubcoreMesh(core_axis_name="c", subcore_axis_name="s")

    @functools.partial(
        pl.kernel,
        mesh=mesh,
        out_type=jax.ShapeDtypeStruct((T, H), jnp.float32),
        scratch_types=[
            pltpu.VMEM((CH,), jnp.int32),
            pltpu.VMEM((CH,), jnp.int32),
            pltpu.VMEM((CH, H), jnp.float32),
            pltpu.VMEM((CH, H), jnp.float32),
            pltpu.VMEM((CH, E), jnp.float32),
            pltpu.VMEM((CH, E), jnp.float32),
        ],
    )
    def k(ys_hbm, p0_hbm, p1_hbm, ws0_hbm, ws1_hbm, out_hbm,
          i0_v, i1_v, r0_v, r1_v, w0_v, w1_v):
        wid = lax.axis_index("s") * NC + lax.axis_index("c")
        base = pl.multiple_of(wid * TPW, TPW)

        @pl.loop(0, TPW // CH)
        def _(cix):
            cbase = pl.multiple_of(base + cix * CH, CH)
            pltpu.sync_copy(p0_hbm.at[pl.ds(cbase, CH)], i0_v)
            pltpu.sync_copy(p1_hbm.at[pl.ds(cbase, CH)], i1_v)
            pltpu.sync_copy(ws0_hbm.at[pl.ds(cbase, CH)], w0_v)
            pltpu.sync_copy(ws1_hbm.at[pl.ds(cbase, CH)], w1_v)
            pltpu.sync_copy(ys_hbm.at[i0_v], r0_v)
            pltpu.sync_copy(ys_hbm.at[i1_v], r1_v)

            @pl.loop(0, CH)
            def _(r):
                w0c = w0_v[r]
                w1c = w1_v[r]

                @pl.loop(0, H // E)
                def _(c):
                    slc = pl.ds(c * E, E)
                    r0_v[r, slc] = r0_v[r, slc] * w0c + r1_v[r, slc] * w1c

            pltpu.sync_copy(r0_v, out_hbm.at[pl.ds(cbase, CH)])

    return k(ys, p0, p1, ws0, ws1)


def kernel(hidden_states, router_logits, w13, w2):
    p0c, p1c, ws0, ws1, sbo, nbo = _routing(router_logits)
    p0 = p0c.reshape(T)
    p1 = p1c.reshape(T)
    sbr = sbo[0]
    nbe = nbo[0]
    xs = _dispatch(hidden_states, p0, p1)
    ys = _grouped_mm(sbr, nbe, xs, w13, w2)
    return _combine(ys, p0, p1, ws0, ws1)


# speedup vs baseline: 2.5122x; 1.0580x over previous
import functools

import jax
import jax.numpy as jnp
from jax import lax
from jax.experimental import pallas as pl
from jax.experimental.pallas import tpu as pltpu
from jax.experimental.pallas import tpu_sc as plsc

T = 2048
H = 1024
E = 16
I = 2816
IT = 1408
NI = I // IT
BLK = 128
NBLK = 48
NROWS = NBLK * BLK
NC = 2
NS = 16
NW = NC * NS
TPW = T // NW
CH = 32


def _routing_body(logits_ref, p0_ref, p1_ref, ws0_ref, ws1_ref, sb_ref, nb_ref):
    lg = logits_ref[...]
    col = lax.broadcasted_iota(jnp.int32, (T, E), 1)
    big = jnp.int32(10**9)
    m1 = jnp.max(lg, axis=1, keepdims=True)
    a1 = jnp.min(jnp.where(lg == m1, col, big), axis=1, keepdims=True)
    lg2 = jnp.where(col == a1, jnp.float32(-1e30), lg)
    m2 = jnp.max(lg2, axis=1, keepdims=True)
    a2 = jnp.min(jnp.where(lg2 == m2, col, big), axis=1, keepdims=True)
    w0 = jax.nn.sigmoid(m1 - m2)
    w1 = 1.0 - w0

    oh = (col == a1).astype(jnp.float32) + (col == a2).astype(jnp.float32)
    r128 = lax.broadcasted_iota(jnp.int32, (128, 128), 0)
    c128 = lax.broadcasted_iota(jnp.int32, (128, 128), 1)
    tril = (r128 > c128).astype(jnp.float32)
    chunks = []
    carry = jnp.zeros((1, E), jnp.float32)
    for k in range(T // 128):
        ch = oh[k * 128:(k + 1) * 128, :]
        chunks.append(jnp.dot(tril, ch, preferred_element_type=jnp.float32) + carry)
        carry = carry + jnp.sum(ch, axis=0, keepdims=True)
    cum = jnp.concatenate(chunks, axis=0)
    counts = carry
    nblk_e = jnp.floor((counts + 127.0) / 128.0)
    r16 = lax.broadcasted_iota(jnp.int32, (E, E), 0)
    c16 = lax.broadcasted_iota(jnp.int32, (E, E), 1)
    upper = (r16 < c16).astype(jnp.float32)
    sb = jnp.dot(nblk_e, upper, preferred_element_type=jnp.float32)
    start = sb * float(BLK)
    pos = jnp.broadcast_to(start, (T, E)) + cum
    p0 = jnp.sum(jnp.where(col == a1, pos, 0.0), axis=1, keepdims=True)
    p1 = jnp.sum(jnp.where(col == a2, pos, 0.0), axis=1, keepdims=True)
    p0_ref[...] = p0.astype(jnp.int32)
    p1_ref[...] = p1.astype(jnp.int32)
    ws0_ref[...] = jnp.broadcast_to(w0, (T, E))
    ws1_ref[...] = jnp.broadcast_to(w1, (T, E))
    sb_ref[...] = jnp.broadcast_to(sb.astype(jnp.int32), (8, E))
    nb_ref[...] = jnp.broadcast_to(nblk_e.astype(jnp.int32), (8, E))


def _routing(router_logits):
    return pl.pallas_call(
        _routing_body,
        out_shape=[
            jax.ShapeDtypeStruct((T, 1), jnp.int32),
            jax.ShapeDtypeStruct((T, 1), jnp.int32),
            jax.ShapeDtypeStruct((T, E), jnp.float32),
            jax.ShapeDtypeStruct((T, E), jnp.float32),
            jax.ShapeDtypeStruct((8, E), jnp.int32),
            jax.ShapeDtypeStruct((8, E), jnp.int32),
        ],
    )(router_logits)


def _dispatch(x, p0, p1):
    mesh = plsc.VectorSubcoreMesh(core_axis_name="c", subcore_axis_name="s")

    @functools.partial(
        pl.kernel,
        mesh=mesh,
        out_type=jax.ShapeDtypeStruct((NROWS, H), jnp.float32),
        scratch_types=[
            pltpu.VMEM((TPW,), jnp.int32),
            pltpu.VMEM((TPW,), jnp.int32),
            pltpu.VMEM((TPW, H), jnp.float32),
        ],
    )
    def k(x_hbm, p0_hbm, p1_hbm, xs_hbm, i0_v, i1_v, rows_v):
        wid = lax.axis_index("s") * NC + lax.axis_index("c")
        base = pl.multiple_of(wid * TPW, TPW)
        pltpu.sync_copy(p0_hbm.at[pl.ds(base, TPW)], i0_v)
        pltpu.sync_copy(p1_hbm.at[pl.ds(base, TPW)], i1_v)
        pltpu.sync_copy(x_hbm.at[pl.ds(base, TPW)], rows_v)
        pltpu.sync_copy(rows_v, xs_hbm.at[i0_v])
        pltpu.sync_copy(rows_v, xs_hbm.at[i1_v])

    return k(x, p0, p1)


MAXB = T // BLK


def _mm_body(sb_ref, nb_ref, xs_ref, wg_ref, wu_ref, w2_ref, ys_ref,
             xb_ref, acc_ref, xsem, fsem):
    e = pl.program_id(0)
    i = pl.program_id(1)

    s0 = sb_ref[e]
    nb = nb_ref[e]
    wgt = wg_ref[0].astype(jnp.bfloat16)
    wut = wu_ref[0].astype(jnp.bfloat16)
    w2t = w2_ref[0].astype(jnp.bfloat16)

    @pl.when(nb > 0)
    def _():
        pltpu.make_async_copy(
            xs_ref.at[pl.ds(s0 * BLK, BLK), :], xb_ref.at[0], xsem).start()

    def blk(j, carry):
        buf = lax.rem(j, 2)
        pltpu.make_async_copy(
            xs_ref.at[pl.ds((s0 + j) * BLK, BLK), :], xb_ref.at[buf],
            xsem).wait()

        @pl.when(j + 1 < nb)
        def _():
            pltpu.make_async_copy(
                xs_ref.at[pl.ds((s0 + j + 1) * BLK, BLK), :],
                xb_ref.at[1 - buf], xsem).start()

        x = xb_ref[buf].astype(jnp.bfloat16)
        g = jnp.dot(x, wgt, preferred_element_type=jnp.float32)
        u = jnp.dot(x, wut, preferred_element_type=jnp.float32)
        h = (g * jax.nn.sigmoid(g) * u).astype(jnp.bfloat16)
        contrib = jnp.dot(h, w2t, preferred_element_type=jnp.float32)
        asl = pl.ds(j * BLK, BLK)

        @pl.when(i == 0)
        def _():
            acc_ref[asl, :] = contrib

        @pl.when(i > 0)
        def _():
            acc_ref[asl, :] += contrib

        return carry

    lax.fori_loop(0, nb, blk, 0)

    @pl.when(i == NI - 1)
    def _():
        def flush(j, carry):
            pltpu.make_async_copy(
                acc_ref.at[pl.ds(j * BLK, BLK), :],
                ys_ref.at[pl.ds((s0 + j) * BLK, BLK), :], fsem).start()
            return carry

        lax.fori_loop(0, nb, flush, 0)

        def drain(j, carry):
            pltpu.make_async_copy(
                acc_ref.at[pl.ds(0, BLK), :],
                ys_ref.at[pl.ds(0, BLK), :], fsem).wait()
            return carry

        lax.fori_loop(0, nb, drain, 0)


def _grouped_mm(sbr, nbe, xs, w13, w2):
    grid_spec = pltpu.PrefetchScalarGridSpec(
        num_scalar_prefetch=2,
        grid=(E, NI),
        in_specs=[
            pl.BlockSpec(memory_space=pl.ANY),
            pl.BlockSpec((1, H, IT), lambda e, i, sb, nb: (e, 0, i)),
            pl.BlockSpec((1, H, IT), lambda e, i, sb, nb: (e, 0, NI + i)),
            pl.BlockSpec((1, IT, H), lambda e, i, sb, nb: (e, i, 0)),
        ],
        out_specs=pl.BlockSpec(memory_space=pl.ANY),
        scratch_shapes=[
            pltpu.VMEM((2, BLK, H), jnp.float32),
            pltpu.VMEM((T, H), jnp.float32),
            pltpu.SemaphoreType.DMA,
            pltpu.SemaphoreType.DMA,
        ],
    )
    return pl.pallas_call(
        _mm_body,
        grid_spec=grid_spec,
        out_shape=jax.ShapeDtypeStruct((NROWS, H), jnp.float32),
        compiler_params=pltpu.CompilerParams(
            dimension_semantics=("arbitrary", "arbitrary"),
            vmem_limit_bytes=63 * 1024 * 1024),
    )(sbr, nbe, xs, w13, w13, w2)


def _combine(ys, p0, p1, ws0, ws1):
    mesh = plsc.VectorSubcoreMesh(core_axis_name="c", subcore_axis_name="s")

    @functools.partial(
        pl.kernel,
        mesh=mesh,
        out_type=jax.ShapeDtypeStruct((T, H), jnp.float32),
        scratch_types=[
            pltpu.VMEM((CH,), jnp.int32),
            pltpu.VMEM((CH,), jnp.int32),
            pltpu.VMEM((CH, H), jnp.float32),
            pltpu.VMEM((CH, H), jnp.float32),
            pltpu.VMEM((CH, E), jnp.float32),
            pltpu.VMEM((CH, E), jnp.float32),
        ],
    )
    def k(ys_hbm, p0_hbm, p1_hbm, ws0_hbm, ws1_hbm, out_hbm,
          i0_v, i1_v, r0_v, r1_v, w0_v, w1_v):
        wid = lax.axis_index("s") * NC + lax.axis_index("c")
        base = pl.multiple_of(wid * TPW, TPW)

        @pl.loop(0, TPW // CH)
        def _(cix):
            cbase = pl.multiple_of(base + cix * CH, CH)
            pltpu.sync_copy(p0_hbm.at[pl.ds(cbase, CH)], i0_v)
            pltpu.sync_copy(p1_hbm.at[pl.ds(cbase, CH)], i1_v)
            pltpu.sync_copy(ws0_hbm.at[pl.ds(cbase, CH)], w0_v)
            pltpu.sync_copy(ws1_hbm.at[pl.ds(cbase, CH)], w1_v)
            pltpu.sync_copy(ys_hbm.at[i0_v], r0_v)
            pltpu.sync_copy(ys_hbm.at[i1_v], r1_v)

            @pl.loop(0, CH)
            def _(r):
                w0c = w0_v[r]
                w1c = w1_v[r]

                @pl.loop(0, H // E)
                def _(c):
                    slc = pl.ds(c * E, E)
                    r0_v[r, slc] = r0_v[r, slc] * w0c + r1_v[r, slc] * w1c

            pltpu.sync_copy(r0_v, out_hbm.at[pl.ds(cbase, CH)])

    return k(ys, p0, p1, ws0, ws1)


def kernel(hidden_states, router_logits, w13, w2):
    p0c, p1c, ws0, ws1, sbo, nbo = _routing(router_logits)
    p0 = p0c.reshape(T)
    p1 = p1c.reshape(T)
    sbr = sbo[0]
    nbe = nbo[0]
    xs = _dispatch(hidden_states, p0, p1)
    ys = _grouped_mm(sbr, nbe, xs, w13, w2)
    return _combine(ys, p0, p1, ws0, ws1)


# speedup vs baseline: 2.5266x; 1.0057x over previous
import functools

import jax
import jax.numpy as jnp
from jax import lax
from jax.experimental import pallas as pl
from jax.experimental.pallas import tpu as pltpu
from jax.experimental.pallas import tpu_sc as plsc

T = 2048
H = 1024
E = 16
I = 2816
IT = 1408
NI = I // IT
BLK = 128
NBLK = 48
NROWS = NBLK * BLK
NC = 2
NS = 16
NW = NC * NS
TPW = T // NW
CH = 32


def _routing_body(logits_ref, p0_ref, p1_ref, ws0_ref, ws1_ref, sb_ref, nb_ref):
    lg = logits_ref[...]
    col = lax.broadcasted_iota(jnp.int32, (T, E), 1)
    big = jnp.int32(10**9)
    m1 = jnp.max(lg, axis=1, keepdims=True)
    a1 = jnp.min(jnp.where(lg == m1, col, big), axis=1, keepdims=True)
    lg2 = jnp.where(col == a1, jnp.float32(-1e30), lg)
    m2 = jnp.max(lg2, axis=1, keepdims=True)
    a2 = jnp.min(jnp.where(lg2 == m2, col, big), axis=1, keepdims=True)
    w0 = jax.nn.sigmoid(m1 - m2)
    w1 = 1.0 - w0

    oh = (col == a1).astype(jnp.float32) + (col == a2).astype(jnp.float32)
    r128 = lax.broadcasted_iota(jnp.int32, (128, 128), 0)
    c128 = lax.broadcasted_iota(jnp.int32, (128, 128), 1)
    tril = (r128 > c128).astype(jnp.float32)
    chunks = []
    carry = jnp.zeros((1, E), jnp.float32)
    for k in range(T // 128):
        ch = oh[k * 128:(k + 1) * 128, :]
        chunks.append(jnp.dot(tril, ch, preferred_element_type=jnp.float32) + carry)
        carry = carry + jnp.sum(ch, axis=0, keepdims=True)
    cum = jnp.concatenate(chunks, axis=0)
    counts = carry
    nblk_e = jnp.floor((counts + 127.0) / 128.0)
    r16 = lax.broadcasted_iota(jnp.int32, (E, E), 0)
    c16 = lax.broadcasted_iota(jnp.int32, (E, E), 1)
    upper = (r16 < c16).astype(jnp.float32)
    sb = jnp.dot(nblk_e, upper, preferred_element_type=jnp.float32)
    start = sb * float(BLK)
    pos = jnp.broadcast_to(start, (T, E)) + cum
    p0 = jnp.sum(jnp.where(col == a1, pos, 0.0), axis=1, keepdims=True)
    p1 = jnp.sum(jnp.where(col == a2, pos, 0.0), axis=1, keepdims=True)
    p0_ref[...] = p0.astype(jnp.int32)
    p1_ref[...] = p1.astype(jnp.int32)
    ws0_ref[...] = jnp.broadcast_to(w0, (T, E))
    ws1_ref[...] = jnp.broadcast_to(w1, (T, E))
    sb_ref[...] = jnp.broadcast_to(sb.astype(jnp.int32), (8, E))
    nb_ref[...] = jnp.broadcast_to(nblk_e.astype(jnp.int32), (8, E))


def _routing(router_logits):
    return pl.pallas_call(
        _routing_body,
        out_shape=[
            jax.ShapeDtypeStruct((T, 1), jnp.int32),
            jax.ShapeDtypeStruct((T, 1), jnp.int32),
            jax.ShapeDtypeStruct((T, E), jnp.float32),
            jax.ShapeDtypeStruct((T, E), jnp.float32),
            jax.ShapeDtypeStruct((8, E), jnp.int32),
            jax.ShapeDtypeStruct((8, E), jnp.int32),
        ],
    )(router_logits)


def _dispatch(x, p0, p1):
    mesh = plsc.VectorSubcoreMesh(core_axis_name="c", subcore_axis_name="s")

    @functools.partial(
        pl.kernel,
        mesh=mesh,
        out_type=jax.ShapeDtypeStruct((NROWS, H), jnp.float32),
        scratch_types=[
            pltpu.VMEM((TPW,), jnp.int32),
            pltpu.VMEM((TPW,), jnp.int32),
            pltpu.VMEM((TPW, H), jnp.float32),
        ],
    )
    def k(x_hbm, p0_hbm, p1_hbm, xs_hbm, i0_v, i1_v, rows_v):
        wid = lax.axis_index("s") * NC + lax.axis_index("c")
        base = pl.multiple_of(wid * TPW, TPW)
        pltpu.sync_copy(p0_hbm.at[pl.ds(base, TPW)], i0_v)
        pltpu.sync_copy(p1_hbm.at[pl.ds(base, TPW)], i1_v)
        pltpu.sync_copy(x_hbm.at[pl.ds(base, TPW)], rows_v)
        pltpu.sync_copy(rows_v, xs_hbm.at[i0_v])
        pltpu.sync_copy(rows_v, xs_hbm.at[i1_v])

    return k(x, p0, p1)


MAXB = T // BLK


def _mm_body(sb_ref, nb_ref, xs_ref, wg_ref, wu_ref, w2_ref, ys_ref,
             xb_ref, acc_ref, xsem, fsem):
    e = pl.program_id(0)
    i = pl.program_id(1)

    s0 = sb_ref[e]
    nb = nb_ref[e]

    @pl.when(nb > 0)
    def _():
        pltpu.make_async_copy(
            xs_ref.at[pl.ds(s0 * BLK, BLK), :], xb_ref.at[0], xsem).start()

    def blk(j, carry):
        buf = lax.rem(j, 2)
        pltpu.make_async_copy(
            xs_ref.at[pl.ds((s0 + j) * BLK, BLK), :], xb_ref.at[buf],
            xsem).wait()

        @pl.when(j + 1 < nb)
        def _():
            pltpu.make_async_copy(
                xs_ref.at[pl.ds((s0 + j + 1) * BLK, BLK), :],
                xb_ref.at[1 - buf], xsem).start()

        x = xb_ref[buf]
        g = jnp.dot(x, wg_ref[0], preferred_element_type=jnp.float32)
        u = jnp.dot(x, wu_ref[0], preferred_element_type=jnp.float32)
        h = g * jax.nn.sigmoid(g) * u
        contrib = jnp.dot(h, w2_ref[0], preferred_element_type=jnp.float32)
        asl = pl.ds(j * BLK, BLK)

        @pl.when(i == 0)
        def _():
            acc_ref[asl, :] = contrib

        @pl.when(i > 0)
        def _():
            acc_ref[asl, :] += contrib

        return carry

    lax.fori_loop(0, nb, blk, 0)

    @pl.when(i == NI - 1)
    def _():
        def flush(j, carry):
            pltpu.make_async_copy(
                acc_ref.at[pl.ds(j * BLK, BLK), :],
                ys_ref.at[pl.ds((s0 + j) * BLK, BLK), :], fsem).start()
            return carry

        lax.fori_loop(0, nb, flush, 0)

        def drain(j, carry):
            pltpu.make_async_copy(
                acc_ref.at[pl.ds(0, BLK), :],
                ys_ref.at[pl.ds(0, BLK), :], fsem).wait()
            return carry

        lax.fori_loop(0, nb, drain, 0)


def _grouped_mm(sbr, nbe, xs, w13, w2):
    grid_spec = pltpu.PrefetchScalarGridSpec(
        num_scalar_prefetch=2,
        grid=(E, NI),
        in_specs=[
            pl.BlockSpec(memory_space=pl.ANY),
            pl.BlockSpec((1, H, IT), lambda e, i, sb, nb: (e, 0, i)),
            pl.BlockSpec((1, H, IT), lambda e, i, sb, nb: (e, 0, NI + i)),
            pl.BlockSpec((1, IT, H), lambda e, i, sb, nb: (e, i, 0)),
        ],
        out_specs=pl.BlockSpec(memory_space=pl.ANY),
        scratch_shapes=[
            pltpu.VMEM((2, BLK, H), jnp.float32),
            pltpu.VMEM((T, H), jnp.float32),
            pltpu.SemaphoreType.DMA,
            pltpu.SemaphoreType.DMA,
        ],
    )
    return pl.pallas_call(
        _mm_body,
        grid_spec=grid_spec,
        out_shape=jax.ShapeDtypeStruct((NROWS, H), jnp.float32),
        compiler_params=pltpu.CompilerParams(
            dimension_semantics=("arbitrary", "arbitrary"),
            vmem_limit_bytes=63 * 1024 * 1024),
    )(sbr, nbe, xs, w13, w13, w2)


def _combine(ys, p0, p1, ws0, ws1):
    mesh = plsc.VectorSubcoreMesh(core_axis_name="c", subcore_axis_name="s")

    @functools.partial(
        pl.kernel,
        mesh=mesh,
        out_type=jax.ShapeDtypeStruct((T, H), jnp.float32),
        scratch_types=[
            pltpu.VMEM((CH,), jnp.int32),
            pltpu.VMEM((CH,), jnp.int32),
            pltpu.VMEM((CH, H), jnp.float32),
            pltpu.VMEM((CH, H), jnp.float32),
            pltpu.VMEM((CH, E), jnp.float32),
            pltpu.VMEM((CH, E), jnp.float32),
        ],
    )
    def k(ys_hbm, p0_hbm, p1_hbm, ws0_hbm, ws1_hbm, out_hbm,
          i0_v, i1_v, r0_v, r1_v, w0_v, w1_v):
        wid = lax.axis_index("s") * NC + lax.axis_index("c")
        base = pl.multiple_of(wid * TPW, TPW)

        @pl.loop(0, TPW // CH)
        def _(cix):
            cbase = pl.multiple_of(base + cix * CH, CH)
            pltpu.sync_copy(p0_hbm.at[pl.ds(cbase, CH)], i0_v)
            pltpu.sync_copy(p1_hbm.at[pl.ds(cbase, CH)], i1_v)
            pltpu.sync_copy(ws0_hbm.at[pl.ds(cbase, CH)], w0_v)
            pltpu.sync_copy(ws1_hbm.at[pl.ds(cbase, CH)], w1_v)
            pltpu.sync_copy(ys_hbm.at[i0_v], r0_v)
            pltpu.sync_copy(ys_hbm.at[i1_v], r1_v)

            @pl.loop(0, CH)
            def _(r):
                w0c = w0_v[r]
                w1c = w1_v[r]

                @pl.loop(0, H // E)
                def _(c):
                    slc = pl.ds(c * E, E)
                    r0_v[r, slc] = r0_v[r, slc] * w0c + r1_v[r, slc] * w1c

            pltpu.sync_copy(r0_v, out_hbm.at[pl.ds(cbase, CH)])

    return k(ys, p0, p1, ws0, ws1)


def kernel(hidden_states, router_logits, w13, w2):
    p0c, p1c, ws0, ws1, sbo, nbo = _routing(router_logits)
    p0 = p0c.reshape(T)
    p1 = p1c.reshape(T)
    sbr = sbo[0]
    nbe = nbo[0]
    xs = _dispatch(hidden_states, p0, p1)
    ys = _grouped_mm(sbr, nbe, xs, w13, w2)
    return _combine(ys, p0, p1, ws0, ws1)
